# baseline (speedup 1.0000x reference)
import functools

import jax
import jax.numpy as jnp
from jax import lax
from jax.experimental import pallas as pl
from jax.experimental.pallas import tpu as pltpu
from jax.experimental.pallas import tpu_sc as plsc

BATCH = 4096
VOCAB = 100000
DIM = 64

_INFO = plsc.get_sparse_core_info()
_NC = _INFO.num_cores
_NS = _INFO.num_subcores
_L = _INFO.num_lanes
_NW = _NC * _NS
_BPW = BATCH // _NW


@functools.partial(
    pl.kernel,
    mesh=plsc.VectorSubcoreMesh(core_axis_name="c", subcore_axis_name="s"),
    out_type=jax.ShapeDtypeStruct((BATCH, DIM), jnp.float32),
    scratch_types=[
        pltpu.VMEM((_BPW,), jnp.int32),
        pltpu.VMEM((_BPW, DIM), jnp.float32),
        pltpu.VMEM((_BPW, DIM), jnp.float32),
        pltpu.SemaphoreType.DMA,
    ],
    compiler_params=pltpu.CompilerParams(use_tc_tiling_on_sc=False),
)
def _gather_add(x_hbm, const_hbm, idx_hbm, out_hbm, idx_v, rows_v, x_v, sem):
    wid = lax.axis_index("s") * _NC + lax.axis_index("c")
    base = wid * _BPW
    pltpu.sync_copy(idx_hbm.at[pl.ds(base, _BPW)], idx_v)
    gather = pltpu.async_copy(const_hbm.at[idx_v], rows_v, sem)
    pltpu.sync_copy(x_hbm.at[pl.ds(base, _BPW), :], x_v)
    gather.wait()

    def body(i, carry):
        for j in range(DIM // _L):
            sl = pl.ds(j * _L, _L)
            rows_v[i, sl] = rows_v[i, sl] + x_v[i, sl]
        return carry

    lax.fori_loop(0, _BPW, body, 0)
    pltpu.sync_copy(rows_v, out_hbm.at[pl.ds(base, _BPW), :])


def kernel(x, const, indices):
    return _gather_add(x, const, indices.astype(jnp.int32))


# speedup vs baseline: 1.2672x; 1.2672x over previous
import functools

import jax
import jax.numpy as jnp
from jax import lax
from jax.experimental import pallas as pl
from jax.experimental.pallas import tpu as pltpu
from jax.experimental.pallas import tpu_sc as plsc

BATCH = 4096
VOCAB = 100000
DIM = 64

_INFO = plsc.get_sparse_core_info()
_NC = _INFO.num_cores
_NS = _INFO.num_subcores
_L = _INFO.num_lanes
_NW = _NC * _NS
_BPW = BATCH // _NW


@functools.partial(
    pl.kernel,
    mesh=plsc.VectorSubcoreMesh(core_axis_name="c", subcore_axis_name="s"),
    out_type=jax.ShapeDtypeStruct((BATCH, DIM), jnp.float32),
    scratch_types=[
        pltpu.VMEM((_BPW,), jnp.int32),
        pltpu.VMEM((_BPW, DIM), jnp.float32),
        pltpu.VMEM((_BPW, DIM), jnp.float32),
        pltpu.SemaphoreType.DMA,
    ],
)
def _gather_add(x_hbm, const_hbm, idx_hbm, out_hbm, idx_v, rows_v, x_v, sem):
    wid = lax.axis_index("s") * _NC + lax.axis_index("c")
    base = wid * _BPW
    pltpu.sync_copy(idx_hbm.at[pl.ds(base, _BPW)], idx_v)
    pltpu.sync_copy(x_hbm.at[pl.ds(base, _BPW), :], x_v)
    for g in range(_BPW // _L):
        vec = idx_v[pl.ds(g * _L, _L)]
        copies = []
        for l in range(_L):
            i = g * _L + l
            cp = pltpu.make_async_copy(
                const_hbm.at[pl.ds(vec[l], 1), :],
                rows_v.at[pl.ds(i, 1), :],
                sem,
            )
            cp.start()
            copies.append(cp)
        for cp in copies:
            cp.wait()

    def body(i, carry):
        for j in range(DIM // _L):
            sl = pl.ds(j * _L, _L)
            rows_v[i, sl] = rows_v[i, sl] + x_v[i, sl]
        return carry

    lax.fori_loop(0, _BPW, body, 0)
    pltpu.sync_copy(rows_v, out_hbm.at[pl.ds(base, _BPW), :])


def kernel(x, const, indices):
    return _gather_add(x, const, indices.astype(jnp.int32))
